# Initial kernel scaffold; baseline (speedup 1.0000x reference)
import jax
import jax.numpy as jnp
from jax.experimental import pallas as pl


def kernel(input_ids, table, W):
    raise NotImplementedError("write your pallas kernel here")


# speedup vs baseline: 15.9596x; 15.9596x over previous
import functools

import jax
import jax.numpy as jnp
from jax import lax
from jax.experimental import pallas as pl
from jax.experimental.pallas import tpu as pltpu
from jax.experimental.pallas import tpu_sc as plsc

VOCAB = 1000000
BOTTLENECK = 32
HIDDEN = 128
B, L = 4096, 200
N = B * L

NC, NS = 2, 16
NW = NC * NS
ROWS_PER_W = N // NW
IDX_W = 128
K_INFLIGHT = 8
CHUNK = K_INFLIGHT * IDX_W
N_OUTER = ROWS_PER_W // CHUNK
IDX_ROWS_PER_W = ROWS_PER_W // IDX_W


def _sc_gather(table, idx2d):
    mesh = plsc.VectorSubcoreMesh(core_axis_name="c", subcore_axis_name="s")

    @functools.partial(
        pl.kernel,
        mesh=mesh,
        compiler_params=pltpu.CompilerParams(use_tc_tiling_on_sc=False),
        out_type=jax.ShapeDtypeStruct((N, BOTTLENECK), jnp.float32),
        scratch_types=[
            pltpu.VMEM((K_INFLIGHT, IDX_W), jnp.int32),
            pltpu.VMEM((CHUNK, BOTTLENECK), jnp.float32),
            pltpu.SemaphoreType.DMA,
        ],
    )
    def k(table_hbm, idx_hbm, out_hbm, idx_v, rows_v, sem):
        wid = lax.axis_index("s") * NC + lax.axis_index("c")

        def body(it, carry):
            idx_row_base = wid * IDX_ROWS_PER_W + it * K_INFLIGHT
            row_base = wid * ROWS_PER_W + it * CHUNK
            pltpu.sync_copy(idx_hbm.at[pl.ds(idx_row_base, K_INFLIGHT)], idx_v)
            handles = []
            for j in range(K_INFLIGHT):
                handles.append(pltpu.async_copy(
                    table_hbm.at[idx_v.at[j]],
                    rows_v.at[pl.ds(j * IDX_W, IDX_W)],
                    sem,
                ))
            for h in handles:
                h.wait()
            pltpu.sync_copy(rows_v, out_hbm.at[pl.ds(row_base, CHUNK)])
            return carry

        lax.fori_loop(0, N_OUTER, body, 0)

    return k(table, idx2d)


def _mm_body(x_ref, w_ref, o_ref):
    o_ref[...] = lax.dot_general(
        x_ref[...], w_ref[...],
        dimension_numbers=(((1,), (1,)), ((), ())),
        preferred_element_type=jnp.float32,
    )


def _tc_expand(compressed, W):
    BM = 2048
    return pl.pallas_call(
        _mm_body,
        grid=(N // BM,),
        in_specs=[
            pl.BlockSpec((BM, BOTTLENECK), lambda i: (i, 0)),
            pl.BlockSpec((HIDDEN, BOTTLENECK), lambda i: (0, 0)),
        ],
        out_specs=pl.BlockSpec((BM, HIDDEN), lambda i: (i, 0)),
        out_shape=jax.ShapeDtypeStruct((N, HIDDEN), jnp.float32),
    )(compressed, W)


def kernel(input_ids, table, W):
    idx2d = input_ids.reshape(N // IDX_W, IDX_W).astype(jnp.int32)
    compressed = _sc_gather(table, idx2d)
    expanded = _tc_expand(compressed, W)
    return expanded.reshape(B, L, HIDDEN)
